# Initial kernel scaffold; baseline (speedup 1.0000x reference)
import functools

import jax
import jax.numpy as jnp
from jax import lax
from jax.experimental import pallas as pl
from jax.experimental.pallas import tpu as pltpu
from jax.experimental.pallas import tpu_sc as plsc

N = 100000
D = 16
E = 3200000
NP = 102400
TPS = NP // 16

NC = 2
NS = 16
NW = NC * NS

CHUNK = 128
NCH = E // CHUNK
CPW = NCH // NW
REM = NCH - CPW * NW
NB = 11
G = CPW // NB

_mesh = plsc.VectorSubcoreMesh(core_axis_name="c", subcore_axis_name="s")


def _worker(c, s):
    wid = s * NC + c
    start = wid * CPW + jnp.minimum(wid, REM)
    return wid, start


def _deg_body(dst2d, zeros1, outa, outb, idxD, ones, dacc, *ssems):
    c = lax.axis_index("c")
    s = lax.axis_index("s")
    wid, start = _worker(c, s)

    for i in range(CHUNK // 16):
        ones[pl.ds(i * 16, 16)] = jnp.ones((16,), jnp.float32)
    sl = pl.ds(s * TPS, TPS)
    pltpu.sync_copy(zeros1, dacc.at[sl])
    plsc.subcore_barrier()

    def grp(g, _):
        row0 = start + g * NB
        pltpu.sync_copy(dst2d.at[pl.ds(row0, NB)], idxD)
        descs = [
            pltpu.async_copy(ones, dacc.at[idxD.at[b]], ssems[b], add=True)
            for b in range(NB)
        ]
        for d_ in descs:
            d_.wait()
        return 0

    lax.fori_loop(0, G, grp, 0)

    @pl.when(wid < REM)
    def _tail():
        row0 = start + G * NB
        pltpu.sync_copy(dst2d.at[pl.ds(row0, 1)], idxD.at[pl.ds(0, 1)])
        pltpu.sync_copy(ones, dacc.at[idxD.at[0]], add=True)

    plsc.subcore_barrier()

    @pl.when(c == 0)
    def _w0():
        pltpu.sync_copy(dacc.at[sl], outa.at[sl])

    @pl.when(c == 1)
    def _w1():
        pltpu.sync_copy(dacc.at[sl], outb.at[sl])


_deg_kernel = pl.kernel(
    _deg_body,
    out_type=(
        jax.ShapeDtypeStruct((NP,), jnp.float32),
        jax.ShapeDtypeStruct((NP,), jnp.float32),
    ),
    mesh=_mesh,
    scratch_types=(
        [pltpu.VMEM((NB, CHUNK), jnp.int32),
         pltpu.VMEM((CHUNK,), jnp.float32),
         pltpu.VMEM_SHARED((NP,), jnp.float32)]
        + [pltpu.SemaphoreType.DMA] * NB
    ),
)


def _gs_body(src2d, dst2d, xs, zeros2, outa, outb, idxS, idxD, rows, acc,
             *sems):
    gsems = sems[:NB]
    ssems = sems[NB:]
    c = lax.axis_index("c")
    s = lax.axis_index("s")
    wid, start = _worker(c, s)

    sl = pl.ds(s * TPS, TPS)
    pltpu.sync_copy(zeros2, acc.at[sl])
    plsc.subcore_barrier()

    def grp(g, _):
        row0 = start + g * NB
        pltpu.sync_copy(src2d.at[pl.ds(row0, NB)], idxS)
        pltpu.sync_copy(dst2d.at[pl.ds(row0, NB)], idxD)
        gds = [
            pltpu.async_copy(xs.at[idxS.at[b]], rows.at[b], gsems[b])
            for b in range(NB)
        ]
        sds = []
        for b in range(NB):
            gds[b].wait()
            sds.append(
                pltpu.async_copy(rows.at[b], acc.at[idxD.at[b]], ssems[b],
                                 add=True))
        for d_ in sds:
            d_.wait()
        return 0

    lax.fori_loop(0, G, grp, 0)

    @pl.when(wid < REM)
    def _tail():
        row0 = start + G * NB
        pltpu.sync_copy(src2d.at[pl.ds(row0, 1)], idxS.at[pl.ds(0, 1)])
        pltpu.sync_copy(dst2d.at[pl.ds(row0, 1)], idxD.at[pl.ds(0, 1)])
        pltpu.sync_copy(xs.at[idxS.at[0]], rows.at[0])
        pltpu.sync_copy(rows.at[0], acc.at[idxD.at[0]], add=True)

    plsc.subcore_barrier()

    @pl.when(c == 0)
    def _w0():
        pltpu.sync_copy(acc.at[sl], outa.at[sl])

    @pl.when(c == 1)
    def _w1():
        pltpu.sync_copy(acc.at[sl], outb.at[sl])


_gs_kernel = pl.kernel(
    _gs_body,
    out_type=(
        jax.ShapeDtypeStruct((NP, D), jnp.float32),
        jax.ShapeDtypeStruct((NP, D), jnp.float32),
    ),
    mesh=_mesh,
    scratch_types=(
        [pltpu.VMEM((NB, CHUNK), jnp.int32),
         pltpu.VMEM((NB, CHUNK), jnp.int32),
         pltpu.VMEM((NB, CHUNK, D), jnp.float32),
         pltpu.VMEM_SHARED((NP, D), jnp.float32)]
        + [pltpu.SemaphoreType.DMA] * (2 * NB)
    ),
)


BM = 5000
GRID = N // BM

_spec_n1 = pl.BlockSpec((BM, 1), lambda i: (i, 0))
_spec_nd = pl.BlockSpec((BM, D), lambda i: (i, 0))
_spec_w = pl.BlockSpec((D, D), lambda i: (0, 0))
_spec_b = pl.BlockSpec((1, D), lambda i: (0, 0))


def _tc1_body(dega, degb, emb, w1, dis, h1, xs1):
    dv = lax.rsqrt(dega[...] + degb[...] + 1.0)
    dis[...] = dv
    h = jnp.dot(emb[...], w1[...], preferred_element_type=jnp.float32)
    h1[...] = h
    xs1[...] = h * dv


_tc1 = pl.pallas_call(
    _tc1_body,
    grid=(GRID,),
    in_specs=[_spec_n1, _spec_n1, _spec_nd, _spec_w],
    out_specs=[_spec_n1, _spec_nd, _spec_nd],
    out_shape=[
        jax.ShapeDtypeStruct((N, 1), jnp.float32),
        jax.ShapeDtypeStruct((N, D), jnp.float32),
        jax.ShapeDtypeStruct((N, D), jnp.float32),
    ],
)


def _tc2_body(dis, h1, acca, accb, w2, bias, h2, xs2):
    dv = dis[...]
    x2 = jnp.maximum(
        dv * (acca[...] + accb[...]) + dv * dv * h1[...] + bias[...], 0.0)
    h = jnp.dot(x2, w2[...], preferred_element_type=jnp.float32)
    h2[...] = h
    xs2[...] = h * dv


_tc2 = pl.pallas_call(
    _tc2_body,
    grid=(GRID,),
    in_specs=[_spec_n1, _spec_nd, _spec_nd, _spec_nd, _spec_w, _spec_b],
    out_specs=[_spec_nd, _spec_nd],
    out_shape=[
        jax.ShapeDtypeStruct((N, D), jnp.float32),
        jax.ShapeDtypeStruct((N, D), jnp.float32),
    ],
)


def _tc3_body(dis, h2, acca, accb, bias, out):
    dv = dis[...]
    out[...] = dv * (acca[...] + accb[...]) + dv * dv * h2[...] + bias[...]


_tc3 = pl.pallas_call(
    _tc3_body,
    grid=(GRID,),
    in_specs=[_spec_n1, _spec_nd, _spec_nd, _spec_nd, _spec_b],
    out_specs=_spec_nd,
    out_shape=jax.ShapeDtypeStruct((N, D), jnp.float32),
)


def kernel(edge_index, emb, W1, b1, W2, b2):
    src2d = edge_index[0].reshape(NCH, CHUNK)
    dst2d = edge_index[1].reshape(NCH, CHUNK)
    zeros1 = jnp.zeros((TPS,), jnp.float32)
    zeros2 = jnp.zeros((TPS, D), jnp.float32)

    dega, degb = _deg_kernel(dst2d, zeros1)
    dis, h1, xs1 = _tc1(dega.reshape(NP, 1), degb.reshape(NP, 1), emb, W1)
    acc1a, acc1b = _gs_kernel(src2d, dst2d, xs1, zeros2)
    h2, xs2 = _tc2(dis, h1, acc1a, acc1b, W2, b1.reshape(1, D))
    acc2a, acc2b = _gs_kernel(src2d, dst2d, xs2, zeros2)
    return _tc3(dis, h2, acc2a, acc2b, b2.reshape(1, D))


# speedup vs baseline: 65.9678x; 65.9678x over previous
import functools

import jax
import jax.numpy as jnp
from jax import lax
from jax.experimental import pallas as pl
from jax.experimental.pallas import tpu as pltpu
from jax.experimental.pallas import tpu_sc as plsc

N = 100000
D = 16
E = 3200000
NP = 102400
TPS = NP // 16

NC = 2
NS = 16
NW = NC * NS

CHUNK = 128
NCH = E // CHUNK
NB = 8
GT = NCH // NB
G = GT // NW
REM = GT - G * NW

_mesh = plsc.VectorSubcoreMesh(core_axis_name="c", subcore_axis_name="s")
_sc_params = pltpu.CompilerParams(use_tc_tiling_on_sc=False)


def _worker(c, s):
    wid = s * NC + c
    start = wid * G + jnp.minimum(wid, REM)
    return wid, start


def _deg_body(dst2d, zeros1, outa, outb, idxD, ones, dacc, *ssems):
    c = lax.axis_index("c")
    s = lax.axis_index("s")
    wid, start = _worker(c, s)

    for i in range(CHUNK // 16):
        ones[pl.ds(i * 16, 16)] = jnp.ones((16,), jnp.float32)
    sl = pl.ds(s * TPS, TPS)
    pltpu.sync_copy(zeros1, dacc.at[sl])
    plsc.subcore_barrier()

    def grp(g, _):
        row0 = (start + g) * NB
        pltpu.sync_copy(dst2d.at[pl.ds(row0, NB)], idxD)
        descs = [
            pltpu.async_copy(ones, dacc.at[idxD.at[b]], ssems[b], add=True)
            for b in range(NB)
        ]
        for d_ in descs:
            d_.wait()
        return 0

    lax.fori_loop(0, G, grp, 0)

    @pl.when(wid < REM)
    def _tail():
        grp(G, 0)

    plsc.subcore_barrier()

    @pl.when(c == 0)
    def _w0():
        pltpu.sync_copy(dacc.at[sl], outa.at[sl])

    @pl.when(c == 1)
    def _w1():
        pltpu.sync_copy(dacc.at[sl], outb.at[sl])


_deg_kernel = pl.kernel(
    _deg_body,
    out_type=(
        jax.ShapeDtypeStruct((NP,), jnp.float32),
        jax.ShapeDtypeStruct((NP,), jnp.float32),
    ),
    mesh=_mesh,
    compiler_params=_sc_params,
    scratch_types=(
        [pltpu.VMEM((NB, CHUNK), jnp.int32),
         pltpu.VMEM((CHUNK,), jnp.float32),
         pltpu.VMEM_SHARED((NP,), jnp.float32)]
        + [pltpu.SemaphoreType.DMA] * NB
    ),
)


def _gs_body(src2d, dst2d, xs, zeros2, outa, outb, idxS, idxD, rows, acc,
             *sems):
    gsems = sems[:NB]
    ssems = sems[NB:]
    c = lax.axis_index("c")
    s = lax.axis_index("s")
    wid, start = _worker(c, s)

    sl = pl.ds(s * TPS, TPS)
    pltpu.sync_copy(zeros2, acc.at[sl])
    plsc.subcore_barrier()

    def grp(g, _):
        row0 = (start + g) * NB
        pltpu.sync_copy(src2d.at[pl.ds(row0, NB)], idxS)
        pltpu.sync_copy(dst2d.at[pl.ds(row0, NB)], idxD)
        gds = [
            pltpu.async_copy(xs.at[idxS.at[b]], rows.at[b], gsems[b])
            for b in range(NB)
        ]
        sds = []
        for b in range(NB):
            gds[b].wait()
            sds.append(
                pltpu.async_copy(rows.at[b], acc.at[idxD.at[b]], ssems[b],
                                 add=True))
        for d_ in sds:
            d_.wait()
        return 0

    lax.fori_loop(0, G, grp, 0)

    @pl.when(wid < REM)
    def _tail():
        grp(G, 0)

    plsc.subcore_barrier()

    @pl.when(c == 0)
    def _w0():
        pltpu.sync_copy(acc.at[sl], outa.at[sl])

    @pl.when(c == 1)
    def _w1():
        pltpu.sync_copy(acc.at[sl], outb.at[sl])


_gs_kernel = pl.kernel(
    _gs_body,
    out_type=(
        jax.ShapeDtypeStruct((NP, D), jnp.float32),
        jax.ShapeDtypeStruct((NP, D), jnp.float32),
    ),
    mesh=_mesh,
    compiler_params=_sc_params,
    scratch_types=(
        [pltpu.VMEM((NB, CHUNK), jnp.int32),
         pltpu.VMEM((NB, CHUNK), jnp.int32),
         pltpu.VMEM((NB, CHUNK, D), jnp.float32),
         pltpu.VMEM_SHARED((NP, D), jnp.float32)]
        + [pltpu.SemaphoreType.DMA] * (2 * NB)
    ),
)


BM = 5000
GRID = N // BM

_spec_n1 = pl.BlockSpec((BM, 1), lambda i: (i, 0))
_spec_nd = pl.BlockSpec((BM, D), lambda i: (i, 0))
_spec_w = pl.BlockSpec((D, D), lambda i: (0, 0))
_spec_b = pl.BlockSpec((1, D), lambda i: (0, 0))


def _tc1_body(dega, degb, emb, w1, dis, h1, xs1):
    dv = lax.rsqrt(dega[...] + degb[...] + 1.0)
    dis[...] = dv
    h = jnp.dot(emb[...], w1[...], preferred_element_type=jnp.float32)
    h1[...] = h
    xs1[...] = h * dv


_tc1 = pl.pallas_call(
    _tc1_body,
    grid=(GRID,),
    in_specs=[_spec_n1, _spec_n1, _spec_nd, _spec_w],
    out_specs=[_spec_n1, _spec_nd, _spec_nd],
    out_shape=[
        jax.ShapeDtypeStruct((N, 1), jnp.float32),
        jax.ShapeDtypeStruct((N, D), jnp.float32),
        jax.ShapeDtypeStruct((N, D), jnp.float32),
    ],
)


def _tc2_body(dis, h1, acca, accb, w2, bias, h2, xs2):
    dv = dis[...]
    x2 = jnp.maximum(
        dv * (acca[...] + accb[...]) + dv * dv * h1[...] + bias[...], 0.0)
    h = jnp.dot(x2, w2[...], preferred_element_type=jnp.float32)
    h2[...] = h
    xs2[...] = h * dv


_tc2 = pl.pallas_call(
    _tc2_body,
    grid=(GRID,),
    in_specs=[_spec_n1, _spec_nd, _spec_nd, _spec_nd, _spec_w, _spec_b],
    out_specs=[_spec_nd, _spec_nd],
    out_shape=[
        jax.ShapeDtypeStruct((N, D), jnp.float32),
        jax.ShapeDtypeStruct((N, D), jnp.float32),
    ],
)


def _tc3_body(dis, h2, acca, accb, bias, out):
    dv = dis[...]
    out[...] = dv * (acca[...] + accb[...]) + dv * dv * h2[...] + bias[...]


_tc3 = pl.pallas_call(
    _tc3_body,
    grid=(GRID,),
    in_specs=[_spec_n1, _spec_nd, _spec_nd, _spec_nd, _spec_b],
    out_specs=_spec_nd,
    out_shape=jax.ShapeDtypeStruct((N, D), jnp.float32),
)


def kernel(edge_index, emb, W1, b1, W2, b2):
    src2d = edge_index[0].reshape(NCH, CHUNK)
    dst2d = edge_index[1].reshape(NCH, CHUNK)
    zeros1 = jnp.zeros((TPS,), jnp.float32)
    zeros2 = jnp.zeros((TPS, D), jnp.float32)

    dega, degb = _deg_kernel(dst2d, zeros1)
    dis, h1, xs1 = _tc1(dega.reshape(NP, 1), degb.reshape(NP, 1), emb, W1)
    acc1a, acc1b = _gs_kernel(src2d, dst2d, xs1, zeros2)
    h2, xs2 = _tc2(dis, h1, acc1a, acc1b, W2, b1.reshape(1, D))
    acc2a, acc2b = _gs_kernel(src2d, dst2d, xs2, zeros2)
    return _tc3(dis, h2, acc2a, acc2b, b2.reshape(1, D))


# speedup vs baseline: 75.8194x; 1.1493x over previous
import jax
import jax.numpy as jnp
from jax import lax
from jax.experimental import pallas as pl
from jax.experimental.pallas import tpu as pltpu
from jax.experimental.pallas import tpu_sc as plsc

N = 100000
D = 16
E = 3200000
NP = 102400
TPS = NP // 16

NC = 2
NS = 16
NW = NC * NS

CHUNK = 128
NB = 4
GPW = 200
SG = 10
SUP = GPW // SG
GTOT = NW * GPW
EP = GTOT * NB * CHUNK
NR = 2
NRD = 2

_mesh = plsc.VectorSubcoreMesh(core_axis_name="c", subcore_axis_name="s")
_sc_params = pltpu.CompilerParams(use_tc_tiling_on_sc=False)


def _deg_body(dst3, zeros1, outa, outb, idxD, ones, dacc, *ssems):
    c = lax.axis_index("c")
    s = lax.axis_index("s")
    wid = s * NC + c
    start = wid * GPW

    for k in range(CHUNK // 16):
        ones[pl.ds(k * 16, 16)] = jnp.ones((16,), jnp.float32)
    sl = pl.ds(s * TPS, TPS)
    pltpu.sync_copy(zeros1, dacc.at[sl])
    plsc.subcore_barrier()

    def scat_wait(u):
        for b in range(NB):
            pltpu.make_async_copy(ones, dacc.at[idxD.at[0, b]],
                                  ssems[u]).wait()

    def scat_fire(u, jj):
        for b in range(NB):
            pltpu.async_copy(ones, dacc.at[idxD.at[jj, b]], ssems[u],
                             add=True)

    def super_body(sg, _):
        @pl.when(sg >= 1)
        def _():
            for u in range(NRD):
                scat_wait(u)
        pltpu.sync_copy(dst3.at[pl.ds(start + sg * SG, SG)], idxD)

        def inner(i, _):
            for u in range(NRD):
                jj = i * NRD + u

                @pl.when(i >= 1)
                def _():
                    scat_wait(u)

                scat_fire(u, jj)
            return 0

        lax.fori_loop(0, SG // NRD, inner, 0)
        return 0

    lax.fori_loop(0, SUP, super_body, 0)
    for u in range(NRD):
        scat_wait(u)

    plsc.subcore_barrier()

    @pl.when(c == 0)
    def _w0():
        pltpu.sync_copy(dacc.at[sl], outa.at[sl])

    @pl.when(c == 1)
    def _w1():
        pltpu.sync_copy(dacc.at[sl], outb.at[sl])


_deg_kernel = pl.kernel(
    _deg_body,
    out_type=(
        jax.ShapeDtypeStruct((NP,), jnp.float32),
        jax.ShapeDtypeStruct((NP,), jnp.float32),
    ),
    mesh=_mesh,
    compiler_params=_sc_params,
    scratch_types=(
        [pltpu.VMEM((SG, NB, CHUNK), jnp.int32),
         pltpu.VMEM((CHUNK,), jnp.float32),
         pltpu.VMEM_SHARED((NP,), jnp.float32)]
        + [pltpu.SemaphoreType.DMA] * NRD
    ),
)


def _gs_body(src3, dst3, xs, zeros2, outa, outb, idxS, idxD,
             r0, r1, acc, *sems):
    rows = (r0, r1)
    gsems = sems[:NR]
    ssems = sems[NR:]
    c = lax.axis_index("c")
    s = lax.axis_index("s")
    wid = s * NC + c
    start = wid * GPW

    sl = pl.ds(s * TPS, TPS)
    pltpu.sync_copy(zeros2, acc.at[sl])
    plsc.subcore_barrier()

    def gath(u, jj):
        return pltpu.make_async_copy(xs.at[idxS.at[jj]], rows[u], gsems[u])

    def scat_wait(u):
        for b in range(NB):
            pltpu.make_async_copy(rows[u].at[pl.ds(b * CHUNK, CHUNK)],
                                  acc.at[idxD.at[0, b]], ssems[u]).wait()

    def scat_fire(u, jj):
        for b in range(NB):
            pltpu.async_copy(rows[u].at[pl.ds(b * CHUNK, CHUNK)],
                             acc.at[idxD.at[jj, b]], ssems[u], add=True)

    def super_body(sg, _):
        @pl.when(sg >= 1)
        def _():
            for u in range(NR):
                scat_wait(u)
        base = start + sg * SG
        pltpu.sync_copy(src3.at[pl.ds(base, SG)], idxS)
        pltpu.sync_copy(dst3.at[pl.ds(base, SG)], idxD)

        def inner(i, _):
            for u in range(NR):
                jj = i * NR + u
                v = (u - 1) % NR

                @pl.when(i >= 1)
                def _():
                    scat_wait(u)

                gath(u, jj).start()

                if u == 0:
                    @pl.when(i >= 1)
                    def _():
                        gath(v, jj - 1).wait()
                        scat_fire(v, jj - 1)
                else:
                    gath(v, jj - 1).wait()
                    scat_fire(v, jj - 1)
            return 0

        lax.fori_loop(0, SG // NR, inner, 0)
        u_last = (SG - 1) % NR
        gath(u_last, SG - 1).wait()
        scat_fire(u_last, SG - 1)
        return 0

    lax.fori_loop(0, SUP, super_body, 0)
    for u in range(NR):
        scat_wait(u)

    plsc.subcore_barrier()

    @pl.when(c == 0)
    def _w0():
        pltpu.sync_copy(acc.at[sl], outa.at[sl])

    @pl.when(c == 1)
    def _w1():
        pltpu.sync_copy(acc.at[sl], outb.at[sl])


_gs_kernel = pl.kernel(
    _gs_body,
    out_type=(
        jax.ShapeDtypeStruct((NP, D), jnp.float32),
        jax.ShapeDtypeStruct((NP, D), jnp.float32),
    ),
    mesh=_mesh,
    compiler_params=_sc_params,
    scratch_types=(
        [pltpu.VMEM((SG, NB * CHUNK), jnp.int32),
         pltpu.VMEM((SG, NB, CHUNK), jnp.int32)]
        + [pltpu.VMEM((NB * CHUNK, D), jnp.float32)] * NR
        + [pltpu.VMEM_SHARED((NP, D), jnp.float32)]
        + [pltpu.SemaphoreType.DMA] * (2 * NR)
    ),
)


BM = 5000
GRID = N // BM

_spec_n1 = pl.BlockSpec((BM, 1), lambda i: (i, 0))
_spec_nd = pl.BlockSpec((BM, D), lambda i: (i, 0))
_spec_w = pl.BlockSpec((D, D), lambda i: (0, 0))
_spec_b = pl.BlockSpec((1, D), lambda i: (0, 0))


def _tc1_body(dega, degb, emb, w1, dis, h1, xs1):
    dv = lax.rsqrt(dega[...] + degb[...] + 1.0)
    dis[...] = dv
    h = jnp.dot(emb[...], w1[...], preferred_element_type=jnp.float32)
    h1[...] = h
    xs1[...] = h * dv


_tc1 = pl.pallas_call(
    _tc1_body,
    grid=(GRID,),
    in_specs=[_spec_n1, _spec_n1, _spec_nd, _spec_w],
    out_specs=[_spec_n1, _spec_nd, _spec_nd],
    out_shape=[
        jax.ShapeDtypeStruct((N, 1), jnp.float32),
        jax.ShapeDtypeStruct((N, D), jnp.float32),
        jax.ShapeDtypeStruct((N, D), jnp.float32),
    ],
)


def _tc2_body(dis, h1, acca, accb, w2, bias, h2, xs2):
    dv = dis[...]
    x2 = jnp.maximum(
        dv * (acca[...] + accb[...]) + dv * dv * h1[...] + bias[...], 0.0)
    h = jnp.dot(x2, w2[...], preferred_element_type=jnp.float32)
    h2[...] = h
    xs2[...] = h * dv


_tc2 = pl.pallas_call(
    _tc2_body,
    grid=(GRID,),
    in_specs=[_spec_n1, _spec_nd, _spec_nd, _spec_nd, _spec_w, _spec_b],
    out_specs=[_spec_nd, _spec_nd],
    out_shape=[
        jax.ShapeDtypeStruct((N, D), jnp.float32),
        jax.ShapeDtypeStruct((N, D), jnp.float32),
    ],
)


def _tc3_body(dis, h2, acca, accb, bias, out):
    dv = dis[...]
    out[...] = dv * (acca[...] + accb[...]) + dv * dv * h2[...] + bias[...]


_tc3 = pl.pallas_call(
    _tc3_body,
    grid=(GRID,),
    in_specs=[_spec_n1, _spec_nd, _spec_nd, _spec_nd, _spec_b],
    out_specs=_spec_nd,
    out_shape=jax.ShapeDtypeStruct((N, D), jnp.float32),
)


def kernel(edge_index, emb, W1, b1, W2, b2):
    pe = EP - E
    pad_src = (jnp.arange(pe, dtype=jnp.int32) % N)
    pad_dst = N + (jnp.arange(pe, dtype=jnp.int32) % 2048)
    src3 = jnp.concatenate([edge_index[0], pad_src]).reshape(GTOT, NB * CHUNK)
    dst3 = jnp.concatenate([edge_index[1], pad_dst]).reshape(GTOT, NB, CHUNK)
    zeros1 = jnp.zeros((TPS,), jnp.float32)
    zeros2 = jnp.zeros((TPS, D), jnp.float32)

    dega, degb = _deg_kernel(dst3, zeros1)
    dis, h1, xs1 = _tc1(dega.reshape(NP, 1), degb.reshape(NP, 1), emb, W1)
    acc1a, acc1b = _gs_kernel(src3, dst3, xs1, zeros2)
    h2, xs2 = _tc2(dis, h1, acc1a, acc1b, W2, b1.reshape(1, D))
    acc2a, acc2b = _gs_kernel(src3, dst3, xs2, zeros2)
    return _tc3(dis, h2, acc2a, acc2b, b2.reshape(1, D))


# speedup vs baseline: 114.7857x; 1.5139x over previous
import jax
import jax.numpy as jnp
from jax import lax
from jax.experimental import pallas as pl
from jax.experimental.pallas import tpu as pltpu
from jax.experimental.pallas import tpu_sc as plsc

N = 100000
D = 16
E = 3200000
NP = 102400
TPS = NP // 16
NP8 = NP // 8

NC = 2
NS = 16
NW = NC * NS

CHUNK = 128
NB = 4
GPW = 200
GPT = 400
SG = 10
GTOT = NW * GPW
EP = GTOT * NB * CHUNK
NR = 2
NRD = 2

HTPS = NP // NC // NS

_mesh = plsc.VectorSubcoreMesh(core_axis_name="c", subcore_axis_name="s")
_sc_params = pltpu.CompilerParams(use_tc_tiling_on_sc=False)


def _deg_body(dst3, zeros1, dis128, idxD, ones, degb, st, dacc, *ssems):
    c = lax.axis_index("c")
    s = lax.axis_index("s")

    for k in range(CHUNK // 16):
        ones[pl.ds(k * 16, 16)] = jnp.ones((16,), jnp.float32)
    sl = pl.ds(s * TPS, TPS)
    pltpu.sync_copy(zeros1, dacc.at[sl])
    plsc.subcore_barrier()

    start = s * GPT

    def scat_wait(u):
        for b in range(NB):
            pltpu.make_async_copy(ones, dacc.at[idxD.at[0, b]],
                                  ssems[u]).wait()

    def scat_fire(u, jj):
        for b in range(NB):
            pltpu.async_copy(ones, dacc.at[idxD.at[jj, b]], ssems[u],
                             add=True)

    def super_body(sg, _):
        @pl.when(sg >= 1)
        def _():
            for u in range(NRD):
                scat_wait(u)
        pltpu.sync_copy(dst3.at[pl.ds(start + sg * SG, SG)], idxD)

        def inner(i, _):
            for u in range(NRD):
                jj = i * NRD + u

                @pl.when(i >= 1)
                def _():
                    scat_wait(u)

                scat_fire(u, jj)
            return 0

        lax.fori_loop(0, SG // NRD, inner, 0)
        return 0

    lax.fori_loop(0, GPT // SG, super_body, 0)
    for u in range(NRD):
        scat_wait(u)

    plsc.subcore_barrier()

    nbase = (c * NS + s) * HTPS
    pltpu.sync_copy(dacc.at[pl.ds(nbase, HTPS)], degb)

    def disv(v, _):
        d = degb[pl.ds(v * 16, 16)] + 1.0
        yi = jnp.int32(0x5F3759DF) - (lax.bitcast_convert_type(
            d, jnp.int32) >> 1)
        y = lax.bitcast_convert_type(yi, jnp.float32)
        for _it in range(3):
            y = y * (1.5 - 0.5 * d * y * y)
        for l in range(16):
            row = 2 * v + (l // 8)
            col = (l % 8) * 16
            st[row, pl.ds(col, 16)] = jnp.broadcast_to(y[l], (16,))
        return 0

    lax.fori_loop(0, HTPS // 16, disv, 0)
    pltpu.sync_copy(st, dis128.at[pl.ds(nbase // 8, HTPS // 8)])


_deg_kernel = pl.kernel(
    _deg_body,
    out_type=jax.ShapeDtypeStruct((NP8, 128), jnp.float32),
    mesh=_mesh,
    compiler_params=_sc_params,
    scratch_types=(
        [pltpu.VMEM((SG, NB, CHUNK), jnp.int32),
         pltpu.VMEM((CHUNK,), jnp.float32),
         pltpu.VMEM((HTPS,), jnp.float32),
         pltpu.VMEM((HTPS // 8, 128), jnp.float32),
         pltpu.VMEM_SHARED((NP,), jnp.float32)]
        + [pltpu.SemaphoreType.DMA] * NRD
    ),
)


def _gs_body(src3, dst3, xs, zeros2, outa, outb, idxS, idxD,
             r0, r1, acc, *sems):
    rows = (r0, r1)
    gsems = sems[:NR]
    ssems = sems[NR:]
    c = lax.axis_index("c")
    s = lax.axis_index("s")
    wid = s * NC + c
    start = wid * GPW

    sl = pl.ds(s * TPS, TPS)
    pltpu.sync_copy(zeros2, acc.at[sl])
    plsc.subcore_barrier()

    def gath(u, jj):
        return pltpu.make_async_copy(xs.at[idxS.at[jj]], rows[u], gsems[u])

    def scat_wait(u):
        for b in range(NB):
            pltpu.make_async_copy(rows[u].at[pl.ds(b * CHUNK, CHUNK)],
                                  acc.at[idxD.at[0, b]], ssems[u]).wait()

    def scat_fire(u, jj):
        for b in range(NB):
            pltpu.async_copy(rows[u].at[pl.ds(b * CHUNK, CHUNK)],
                             acc.at[idxD.at[jj, b]], ssems[u], add=True)

    def super_body(sg, _):
        @pl.when(sg >= 1)
        def _():
            for u in range(NR):
                scat_wait(u)
        base = start + sg * SG
        pltpu.sync_copy(src3.at[pl.ds(base, SG)], idxS)
        pltpu.sync_copy(dst3.at[pl.ds(base, SG)], idxD)

        def inner(i, _):
            for u in range(NR):
                jj = i * NR + u
                v = (u - 1) % NR

                @pl.when(i >= 1)
                def _():
                    scat_wait(u)

                gath(u, jj).start()

                if u == 0:
                    @pl.when(i >= 1)
                    def _():
                        gath(v, jj - 1).wait()
                        scat_fire(v, jj - 1)
                else:
                    gath(v, jj - 1).wait()
                    scat_fire(v, jj - 1)
            return 0

        lax.fori_loop(0, SG // NR, inner, 0)
        u_last = (SG - 1) % NR
        gath(u_last, SG - 1).wait()
        scat_fire(u_last, SG - 1)
        return 0

    lax.fori_loop(0, GPW // SG, super_body, 0)
    for u in range(NR):
        scat_wait(u)

    plsc.subcore_barrier()

    @pl.when(c == 0)
    def _w0():
        pltpu.sync_copy(acc.at[sl], outa.at[sl])

    @pl.when(c == 1)
    def _w1():
        pltpu.sync_copy(acc.at[sl], outb.at[sl])


_gs_kernel = pl.kernel(
    _gs_body,
    out_type=(
        jax.ShapeDtypeStruct((NP, D), jnp.float32),
        jax.ShapeDtypeStruct((NP, D), jnp.float32),
    ),
    mesh=_mesh,
    compiler_params=_sc_params,
    scratch_types=(
        [pltpu.VMEM((SG, NB * CHUNK), jnp.int32),
         pltpu.VMEM((SG, NB, CHUNK), jnp.int32)]
        + [pltpu.VMEM((NB * CHUNK, D), jnp.float32)] * NR
        + [pltpu.VMEM_SHARED((NP, D), jnp.float32)]
        + [pltpu.SemaphoreType.DMA] * (2 * NR)
    ),
)


BM8 = 800
GRID = NP8 // BM8

_spec_p = pl.BlockSpec((BM8, 128), lambda i: (i, 0))
_spec_w = pl.BlockSpec((128, 128), lambda i: (0, 0))
_spec_b = pl.BlockSpec((1, 128), lambda i: (0, 0))


def _tc1_body(dis, emb, wbd, xs1):
    h = jnp.dot(emb[...], wbd[...], preferred_element_type=jnp.float32)
    xs1[...] = h * dis[...]


_tc1 = pl.pallas_call(
    _tc1_body,
    grid=(GRID,),
    in_specs=[_spec_p, _spec_p, _spec_w],
    out_specs=_spec_p,
    out_shape=jax.ShapeDtypeStruct((NP8, 128), jnp.float32),
)


def _tc2_body(dis, xs1, acca, accb, wbd, bias, xs2):
    dv = dis[...]
    x2 = jnp.maximum(
        dv * (acca[...] + accb[...] + xs1[...]) + bias[...], 0.0)
    h = jnp.dot(x2, wbd[...], preferred_element_type=jnp.float32)
    xs2[...] = h * dv


_tc2 = pl.pallas_call(
    _tc2_body,
    grid=(GRID,),
    in_specs=[_spec_p, _spec_p, _spec_p, _spec_p, _spec_w, _spec_b],
    out_specs=_spec_p,
    out_shape=jax.ShapeDtypeStruct((NP8, 128), jnp.float32),
)


def _tc3_body(dis, xs2, acca, accb, bias, out):
    out[...] = dis[...] * (acca[...] + accb[...] + xs2[...]) + bias[...]


_tc3 = pl.pallas_call(
    _tc3_body,
    grid=(GRID,),
    in_specs=[_spec_p, _spec_p, _spec_p, _spec_p, _spec_b],
    out_specs=_spec_p,
    out_shape=jax.ShapeDtypeStruct((NP8, 128), jnp.float32),
)


def kernel(edge_index, emb, W1, b1, W2, b2):
    pe = EP - E
    pad_src = (jnp.arange(pe, dtype=jnp.int32) % N)
    pad_dst = N + (jnp.arange(pe, dtype=jnp.int32) % 2048)
    src3 = jnp.concatenate([edge_index[0], pad_src]).reshape(GTOT, NB * CHUNK)
    dst3 = jnp.concatenate([edge_index[1], pad_dst]).reshape(GTOT, NB, CHUNK)
    zeros1 = jnp.zeros((TPS,), jnp.float32)
    zeros2 = jnp.zeros((TPS, D), jnp.float32)

    eye8 = jnp.eye(8, dtype=jnp.float32)
    wbd1 = jnp.kron(eye8, W1)
    wbd2 = jnp.kron(eye8, W2)
    b1t = jnp.tile(b1, 8).reshape(1, 128)
    b2t = jnp.tile(b2, 8).reshape(1, 128)
    emb8 = jnp.pad(emb.reshape(N // 8, 128), ((0, NP8 - N // 8), (0, 0)))

    dis = _deg_kernel(dst3, zeros1)
    xs1 = _tc1(dis, emb8, wbd1)
    a1a, a1b = _gs_kernel(src3, dst3, xs1.reshape(NP, D), zeros2)
    xs2 = _tc2(dis, xs1, a1a.reshape(NP8, 128), a1b.reshape(NP8, 128),
               wbd2, b1t)
    a2a, a2b = _gs_kernel(src3, dst3, xs2.reshape(NP, D), zeros2)
    out = _tc3(dis, xs2, a2a.reshape(NP8, 128), a2b.reshape(NP8, 128), b2t)
    return out[:N // 8].reshape(N, D)


# speedup vs baseline: 115.0412x; 1.0022x over previous
import jax
import jax.numpy as jnp
from jax import lax
from jax.experimental import pallas as pl
from jax.experimental.pallas import tpu as pltpu
from jax.experimental.pallas import tpu_sc as plsc

N = 100000
D = 16
E = 3200000
NP = 102400
TPS = NP // 16
NP8 = NP // 8

NC = 2
NS = 16
NW = NC * NS

CHUNK = 128
NB = 4
GPW = 200
GPT = 400
SG = 10
GTOT = NW * GPW
EP = GTOT * NB * CHUNK
NR = 2
NRD = 2

HTPS = NP // NC // NS

_mesh = plsc.VectorSubcoreMesh(core_axis_name="c", subcore_axis_name="s")
_sc_params = pltpu.CompilerParams(use_tc_tiling_on_sc=False)


def _deg_body(dst3, zeros1, dis128, idxD, ones, degb, st, dacc, *ssems):
    c = lax.axis_index("c")
    s = lax.axis_index("s")

    for k in range(NB * CHUNK // 16):
        ones[pl.ds(k * 16, 16)] = jnp.ones((16,), jnp.float32)
    sl = pl.ds(s * TPS, TPS)
    pltpu.sync_copy(zeros1, dacc.at[sl])
    plsc.subcore_barrier()

    start = s * GPT

    def scat_wait(u):
        pltpu.make_async_copy(ones, dacc.at[idxD.at[0]], ssems[u]).wait()

    def scat_fire(u, jj):
        pltpu.async_copy(ones, dacc.at[idxD.at[jj]], ssems[u], add=True)

    def super_body(sg, _):
        @pl.when(sg >= 1)
        def _():
            for u in range(NRD):
                scat_wait(u)
        pltpu.sync_copy(dst3.at[pl.ds(start + sg * SG, SG)], idxD)

        def inner(i, _):
            for u in range(NRD):
                jj = i * NRD + u

                @pl.when(i >= 1)
                def _():
                    scat_wait(u)

                scat_fire(u, jj)
            return 0

        lax.fori_loop(0, SG // NRD, inner, 0)
        return 0

    lax.fori_loop(0, GPT // SG, super_body, 0)
    for u in range(NRD):
        scat_wait(u)

    plsc.subcore_barrier()

    nbase = (c * NS + s) * HTPS
    pltpu.sync_copy(dacc.at[pl.ds(nbase, HTPS)], degb)

    def disv(v, _):
        d = degb[pl.ds(v * 16, 16)] + 1.0
        yi = jnp.int32(0x5F3759DF) - (lax.bitcast_convert_type(
            d, jnp.int32) >> 1)
        y = lax.bitcast_convert_type(yi, jnp.float32)
        for _it in range(3):
            y = y * (1.5 - 0.5 * d * y * y)
        for l in range(16):
            row = 2 * v + (l // 8)
            col = (l % 8) * 16
            st[row, pl.ds(col, 16)] = jnp.broadcast_to(y[l], (16,))
        return 0

    lax.fori_loop(0, HTPS // 16, disv, 0)
    pltpu.sync_copy(st, dis128.at[pl.ds(nbase // 8, HTPS // 8)])


_deg_kernel = pl.kernel(
    _deg_body,
    out_type=jax.ShapeDtypeStruct((NP8, 128), jnp.float32),
    mesh=_mesh,
    compiler_params=_sc_params,
    scratch_types=(
        [pltpu.VMEM((SG, NB * CHUNK), jnp.int32),
         pltpu.VMEM((NB * CHUNK,), jnp.float32),
         pltpu.VMEM((HTPS,), jnp.float32),
         pltpu.VMEM((HTPS // 8, 128), jnp.float32),
         pltpu.VMEM_SHARED((NP,), jnp.float32)]
        + [pltpu.SemaphoreType.DMA] * NRD
    ),
)


def _gs_body(src3, dst3, xs, zeros2, outa, outb, idxS, idxD,
             r0, r1, acc, *sems):
    rows = (r0, r1)
    gsems = sems[:NR]
    ssems = sems[NR:]
    c = lax.axis_index("c")
    s = lax.axis_index("s")
    wid = s * NC + c
    start = wid * GPW

    sl = pl.ds(s * TPS, TPS)
    pltpu.sync_copy(zeros2, acc.at[sl])
    plsc.subcore_barrier()

    def gath(u, jj):
        return pltpu.make_async_copy(xs.at[idxS.at[jj]], rows[u], gsems[u])

    def scat_wait(u):
        pltpu.make_async_copy(rows[u], acc.at[idxD.at[0]], ssems[u]).wait()

    def scat_fire(u, jj):
        pltpu.async_copy(rows[u], acc.at[idxD.at[jj]], ssems[u], add=True)

    def super_body(sg, _):
        @pl.when(sg >= 1)
        def _():
            for u in range(NR):
                scat_wait(u)
        base = start + sg * SG
        pltpu.sync_copy(src3.at[pl.ds(base, SG)], idxS)
        pltpu.sync_copy(dst3.at[pl.ds(base, SG)], idxD)

        def inner(i, _):
            for u in range(NR):
                jj = i * NR + u
                v = (u - 1) % NR

                @pl.when(i >= 1)
                def _():
                    scat_wait(u)

                gath(u, jj).start()

                if u == 0:
                    @pl.when(i >= 1)
                    def _():
                        gath(v, jj - 1).wait()
                        scat_fire(v, jj - 1)
                else:
                    gath(v, jj - 1).wait()
                    scat_fire(v, jj - 1)
            return 0

        lax.fori_loop(0, SG // NR, inner, 0)
        u_last = (SG - 1) % NR
        gath(u_last, SG - 1).wait()
        scat_fire(u_last, SG - 1)
        return 0

    lax.fori_loop(0, GPW // SG, super_body, 0)
    for u in range(NR):
        scat_wait(u)

    plsc.subcore_barrier()

    @pl.when(c == 0)
    def _w0():
        pltpu.sync_copy(acc.at[sl], outa.at[sl])

    @pl.when(c == 1)
    def _w1():
        pltpu.sync_copy(acc.at[sl], outb.at[sl])


_gs_kernel = pl.kernel(
    _gs_body,
    out_type=(
        jax.ShapeDtypeStruct((NP, D), jnp.float32),
        jax.ShapeDtypeStruct((NP, D), jnp.float32),
    ),
    mesh=_mesh,
    compiler_params=_sc_params,
    scratch_types=(
        [pltpu.VMEM((SG, NB * CHUNK), jnp.int32),
         pltpu.VMEM((SG, NB * CHUNK), jnp.int32)]
        + [pltpu.VMEM((NB * CHUNK, D), jnp.float32)] * NR
        + [pltpu.VMEM_SHARED((NP, D), jnp.float32)]
        + [pltpu.SemaphoreType.DMA] * (2 * NR)
    ),
)


BM8 = 800
GRID = NP8 // BM8

_spec_p = pl.BlockSpec((BM8, 128), lambda i: (i, 0))
_spec_w = pl.BlockSpec((128, 128), lambda i: (0, 0))
_spec_b = pl.BlockSpec((1, 128), lambda i: (0, 0))


def _tc1_body(dis, emb, wbd, xs1):
    h = jnp.dot(emb[...], wbd[...], preferred_element_type=jnp.float32)
    xs1[...] = h * dis[...]


_tc1 = pl.pallas_call(
    _tc1_body,
    grid=(GRID,),
    in_specs=[_spec_p, _spec_p, _spec_w],
    out_specs=_spec_p,
    out_shape=jax.ShapeDtypeStruct((NP8, 128), jnp.float32),
)


def _tc2_body(dis, xs1, acca, accb, wbd, bias, xs2):
    dv = dis[...]
    x2 = jnp.maximum(
        dv * (acca[...] + accb[...] + xs1[...]) + bias[...], 0.0)
    h = jnp.dot(x2, wbd[...], preferred_element_type=jnp.float32)
    xs2[...] = h * dv


_tc2 = pl.pallas_call(
    _tc2_body,
    grid=(GRID,),
    in_specs=[_spec_p, _spec_p, _spec_p, _spec_p, _spec_w, _spec_b],
    out_specs=_spec_p,
    out_shape=jax.ShapeDtypeStruct((NP8, 128), jnp.float32),
)


def _tc3_body(dis, xs2, acca, accb, bias, out):
    out[...] = dis[...] * (acca[...] + accb[...] + xs2[...]) + bias[...]


_tc3 = pl.pallas_call(
    _tc3_body,
    grid=(GRID,),
    in_specs=[_spec_p, _spec_p, _spec_p, _spec_p, _spec_b],
    out_specs=_spec_p,
    out_shape=jax.ShapeDtypeStruct((NP8, 128), jnp.float32),
)


def kernel(edge_index, emb, W1, b1, W2, b2):
    pe = EP - E
    pad_src = (jnp.arange(pe, dtype=jnp.int32) % N)
    pad_dst = N + (jnp.arange(pe, dtype=jnp.int32) % 2048)
    src3 = jnp.concatenate([edge_index[0], pad_src]).reshape(GTOT, NB * CHUNK)
    dst3 = jnp.concatenate([edge_index[1], pad_dst]).reshape(GTOT, NB * CHUNK)
    zeros1 = jnp.zeros((TPS,), jnp.float32)
    zeros2 = jnp.zeros((TPS, D), jnp.float32)

    eye8 = jnp.eye(8, dtype=jnp.float32)
    wbd1 = jnp.kron(eye8, W1)
    wbd2 = jnp.kron(eye8, W2)
    b1t = jnp.tile(b1, 8).reshape(1, 128)
    b2t = jnp.tile(b2, 8).reshape(1, 128)
    emb8 = jnp.pad(emb.reshape(N // 8, 128), ((0, NP8 - N // 8), (0, 0)))

    dis = _deg_kernel(dst3, zeros1)
    xs1 = _tc1(dis, emb8, wbd1)
    a1a, a1b = _gs_kernel(src3, dst3, xs1.reshape(NP, D), zeros2)
    xs2 = _tc2(dis, xs1, a1a.reshape(NP8, 128), a1b.reshape(NP8, 128),
               wbd2, b1t)
    a2a, a2b = _gs_kernel(src3, dst3, xs2.reshape(NP, D), zeros2)
    out = _tc3(dis, xs2, a2a.reshape(NP8, 128), a2b.reshape(NP8, 128), b2t)
    return out[:N // 8].reshape(N, D)


# speedup vs baseline: 123.0864x; 1.0699x over previous
import jax
import jax.numpy as jnp
from jax import lax
from jax.experimental import pallas as pl
from jax.experimental.pallas import tpu as pltpu
from jax.experimental.pallas import tpu_sc as plsc

N = 100000
D = 16
E = 3200000
NP = 102400
TPS = NP // 16
NP8 = NP // 8

NC = 2
NS = 16
NW = NC * NS

GW = 256
GPW = 400
GPT = 800
GTOT = NW * GPW
EP = GTOT * GW
NR = 4

HTPS = NP // NC // NS

_mesh = plsc.VectorSubcoreMesh(core_axis_name="c", subcore_axis_name="s")
_sc_params = pltpu.CompilerParams(use_tc_tiling_on_sc=False)


def _deg_body(dst3, zeros1, dis128, idxD, ones, degb, st, dacc, *sems):
    dsems = sems[:NR]
    ssems = sems[NR:]
    c = lax.axis_index("c")
    s = lax.axis_index("s")

    for k in range(GW // 16):
        ones[pl.ds(k * 16, 16)] = jnp.ones((16,), jnp.float32)
    sl = pl.ds(s * TPS, TPS)
    pltpu.sync_copy(zeros1, dacc.at[sl])
    plsc.subcore_barrier()

    start = s * GPT

    def dld(q, g):
        return pltpu.make_async_copy(dst3.at[g], idxD.at[q], dsems[q])

    def scat(q):
        return pltpu.make_async_copy(ones, dacc.at[idxD.at[q]], ssems[q])

    def body(i, _):
        for u in range(NR):
            w = (u + 2) % NR

            @pl.when(i >= 1)
            def _():
                scat(u).wait()

            dld(u, start + i * NR + u).start()
            if u >= 2:
                dld(w, 0).wait()
                pltpu.async_copy(ones, dacc.at[idxD.at[w]], ssems[w],
                                 add=True)
            else:
                @pl.when(i >= 1)
                def _():
                    dld(w, 0).wait()
                    pltpu.async_copy(ones, dacc.at[idxD.at[w]], ssems[w],
                                     add=True)
        return 0

    lax.fori_loop(0, GPT // NR, body, 0)
    for u in (2, 3):
        dld(u, 0).wait()
        pltpu.async_copy(ones, dacc.at[idxD.at[u]], ssems[u], add=True)
    for u in range(NR):
        scat(u).wait()

    plsc.subcore_barrier()

    nbase = (c * NS + s) * HTPS
    pltpu.sync_copy(dacc.at[pl.ds(nbase, HTPS)], degb)

    def disv(v, _):
        d = degb[pl.ds(v * 16, 16)] + 1.0
        yi = jnp.int32(0x5F3759DF) - (lax.bitcast_convert_type(
            d, jnp.int32) >> 1)
        y = lax.bitcast_convert_type(yi, jnp.float32)
        for _it in range(3):
            y = y * (1.5 - 0.5 * d * y * y)
        for l in range(16):
            row = 2 * v + (l // 8)
            col = (l % 8) * 16
            st[row, pl.ds(col, 16)] = jnp.broadcast_to(y[l], (16,))
        return 0

    lax.fori_loop(0, HTPS // 16, disv, 0)
    pltpu.sync_copy(st, dis128.at[pl.ds(nbase // 8, HTPS // 8)])


_deg_kernel = pl.kernel(
    _deg_body,
    out_type=jax.ShapeDtypeStruct((NP8, 128), jnp.float32),
    mesh=_mesh,
    compiler_params=_sc_params,
    scratch_types=(
        [pltpu.VMEM((NR, GW), jnp.int32),
         pltpu.VMEM((GW,), jnp.float32),
         pltpu.VMEM((HTPS,), jnp.float32),
         pltpu.VMEM((HTPS // 8, 128), jnp.float32),
         pltpu.VMEM_SHARED((NP,), jnp.float32)]
        + [pltpu.SemaphoreType.DMA] * (2 * NR)
    ),
)


def _gs_body(src3, dst3, xs, zeros2, outa, outb, idxS, idxD,
             r0, r1, r2, r3, acc, *sems):
    rows = (r0, r1, r2, r3)
    isems = sems[:NR]
    dsems = sems[NR:2 * NR]
    gsems = sems[2 * NR:3 * NR]
    ssems = sems[3 * NR:]
    c = lax.axis_index("c")
    s = lax.axis_index("s")
    wid = s * NC + c
    start = wid * GPW

    sl = pl.ds(s * TPS, TPS)
    pltpu.sync_copy(zeros2, acc.at[sl])
    plsc.subcore_barrier()

    def ild(q, g):
        return pltpu.make_async_copy(src3.at[g], idxS.at[q], isems[q])

    def dld(q, g):
        return pltpu.make_async_copy(dst3.at[g], idxD.at[q], dsems[q])

    def gath(q):
        return pltpu.make_async_copy(xs.at[idxS.at[q]], rows[q], gsems[q])

    def scat(q):
        return pltpu.make_async_copy(rows[q], acc.at[idxD.at[q]], ssems[q])

    ild(0, start).start()
    ild(1, start + 1).start()

    def body(i, _):
        for u in range(NR):
            w = (u + 2) % NR
            g = start + i * NR + u

            @pl.when(i >= 1)
            def _():
                scat(u).wait()

            dld(u, g).start()
            ild(u, 0).wait()
            gath(u).start()

            def fire_prev():
                gath(w).wait()
                dld(w, 0).wait()
                pltpu.async_copy(rows[w], acc.at[idxD.at[w]], ssems[w],
                                 add=True)

            if u >= 2:
                fire_prev()

                @pl.when(i <= GPW // NR - 2)
                def _():
                    ild(w, g + 2).start()
            else:
                @pl.when(i >= 1)
                def _():
                    fire_prev()

                ild(w, g + 2).start()
        return 0

    lax.fori_loop(0, GPW // NR, body, 0)
    for u in (2, 3):
        gath(u).wait()
        dld(u, 0).wait()
        pltpu.async_copy(rows[u], acc.at[idxD.at[u]], ssems[u], add=True)
    for u in range(NR):
        scat(u).wait()

    plsc.subcore_barrier()

    @pl.when(c == 0)
    def _w0():
        pltpu.sync_copy(acc.at[sl], outa.at[sl])

    @pl.when(c == 1)
    def _w1():
        pltpu.sync_copy(acc.at[sl], outb.at[sl])


_gs_kernel = pl.kernel(
    _gs_body,
    out_type=(
        jax.ShapeDtypeStruct((NP, D), jnp.float32),
        jax.ShapeDtypeStruct((NP, D), jnp.float32),
    ),
    mesh=_mesh,
    compiler_params=_sc_params,
    scratch_types=(
        [pltpu.VMEM((NR, GW), jnp.int32),
         pltpu.VMEM((NR, GW), jnp.int32)]
        + [pltpu.VMEM((GW, D), jnp.float32)] * NR
        + [pltpu.VMEM_SHARED((NP, D), jnp.float32)]
        + [pltpu.SemaphoreType.DMA] * (4 * NR)
    ),
)


BM8 = 800
GRID = NP8 // BM8

_spec_p = pl.BlockSpec((BM8, 128), lambda i: (i, 0))
_spec_w = pl.BlockSpec((128, 128), lambda i: (0, 0))
_spec_b = pl.BlockSpec((1, 128), lambda i: (0, 0))


def _tc1_body(dis, emb, wbd, xs1):
    h = jnp.dot(emb[...], wbd[...], preferred_element_type=jnp.float32)
    xs1[...] = h * dis[...]


_tc1 = pl.pallas_call(
    _tc1_body,
    grid=(GRID,),
    in_specs=[_spec_p, _spec_p, _spec_w],
    out_specs=_spec_p,
    out_shape=jax.ShapeDtypeStruct((NP8, 128), jnp.float32),
)


def _tc2_body(dis, xs1, acca, accb, wbd, bias, xs2):
    dv = dis[...]
    x2 = jnp.maximum(
        dv * (acca[...] + accb[...] + xs1[...]) + bias[...], 0.0)
    h = jnp.dot(x2, wbd[...], preferred_element_type=jnp.float32)
    xs2[...] = h * dv


_tc2 = pl.pallas_call(
    _tc2_body,
    grid=(GRID,),
    in_specs=[_spec_p, _spec_p, _spec_p, _spec_p, _spec_w, _spec_b],
    out_specs=_spec_p,
    out_shape=jax.ShapeDtypeStruct((NP8, 128), jnp.float32),
)


def _tc3_body(dis, xs2, acca, accb, bias, out):
    out[...] = dis[...] * (acca[...] + accb[...] + xs2[...]) + bias[...]


_tc3 = pl.pallas_call(
    _tc3_body,
    grid=(GRID,),
    in_specs=[_spec_p, _spec_p, _spec_p, _spec_p, _spec_b],
    out_specs=_spec_p,
    out_shape=jax.ShapeDtypeStruct((NP8, 128), jnp.float32),
)


def kernel(edge_index, emb, W1, b1, W2, b2):
    pe = EP - E
    pad_src = (jnp.arange(pe, dtype=jnp.int32) % N)
    pad_dst = N + (jnp.arange(pe, dtype=jnp.int32) % 2048)
    src3 = jnp.concatenate([edge_index[0], pad_src]).reshape(GTOT, GW)
    dst3 = jnp.concatenate([edge_index[1], pad_dst]).reshape(GTOT, GW)
    zeros1 = jnp.zeros((TPS,), jnp.float32)
    zeros2 = jnp.zeros((TPS, D), jnp.float32)

    eye8 = jnp.eye(8, dtype=jnp.float32)
    wbd1 = jnp.kron(eye8, W1)
    wbd2 = jnp.kron(eye8, W2)
    b1t = jnp.tile(b1, 8).reshape(1, 128)
    b2t = jnp.tile(b2, 8).reshape(1, 128)
    emb8 = jnp.pad(emb.reshape(N // 8, 128), ((0, NP8 - N // 8), (0, 0)))

    dis = _deg_kernel(dst3, zeros1)
    xs1 = _tc1(dis, emb8, wbd1)
    a1a, a1b = _gs_kernel(src3, dst3, xs1.reshape(NP, D), zeros2)
    xs2 = _tc2(dis, xs1, a1a.reshape(NP8, 128), a1b.reshape(NP8, 128),
               wbd2, b1t)
    a2a, a2b = _gs_kernel(src3, dst3, xs2.reshape(NP, D), zeros2)
    out = _tc3(dis, xs2, a2a.reshape(NP8, 128), a2b.reshape(NP8, 128), b2t)
    return out[:N // 8].reshape(N, D)


# speedup vs baseline: 142.0317x; 1.1539x over previous
import jax
import jax.numpy as jnp
from jax import lax
from jax.experimental import pallas as pl
from jax.experimental.pallas import tpu as pltpu
from jax.experimental.pallas import tpu_sc as plsc

N = 100000
D = 16
E = 3200000
NP = 102400
TPS = NP // 16
NP8 = NP // 8

NC = 2
NS = 16
NW = NC * NS

GW = 256
GPW = 400
GTOT = NW * GPW
EP = GTOT * GW
NR = 4
GWD = 2048
GPT = EP // GWD // NS

HTPS = NP // NC // NS

_mesh = plsc.VectorSubcoreMesh(core_axis_name="c", subcore_axis_name="s")
_sc_params = pltpu.CompilerParams(use_tc_tiling_on_sc=False)


def _deg_body(dst3, zeros1, dis128, idxD, ones, degb, st, dacc, *sems):
    dsems = sems[:NR]
    ssems = sems[NR:]
    c = lax.axis_index("c")
    s = lax.axis_index("s")

    for k in range(GWD // 16):
        ones[pl.ds(k * 16, 16)] = jnp.ones((16,), jnp.float32)
    sl = pl.ds(s * TPS, TPS)
    pltpu.sync_copy(zeros1, dacc.at[sl])
    plsc.subcore_barrier()

    start = s * GPT

    def dld(q, g):
        return pltpu.make_async_copy(dst3.at[g], idxD.at[q], dsems[q])

    def scat(q):
        return pltpu.make_async_copy(ones, dacc.at[idxD.at[q]], ssems[q])

    def body(i, _):
        for u in range(NR):
            w = (u + 2) % NR

            @pl.when(i >= 1)
            def _():
                scat(u).wait()

            dld(u, start + i * NR + u).start()
            if u >= 2:
                dld(w, 0).wait()
                pltpu.async_copy(ones, dacc.at[idxD.at[w]], ssems[w],
                                 add=True)
            else:
                @pl.when(i >= 1)
                def _():
                    dld(w, 0).wait()
                    pltpu.async_copy(ones, dacc.at[idxD.at[w]], ssems[w],
                                     add=True)
        return 0

    lax.fori_loop(0, GPT // NR, body, 0)
    for u in (2, 3):
        dld(u, 0).wait()
        pltpu.async_copy(ones, dacc.at[idxD.at[u]], ssems[u], add=True)
    for u in range(NR):
        scat(u).wait()

    plsc.subcore_barrier()

    nbase = (c * NS + s) * HTPS
    pltpu.sync_copy(dacc.at[pl.ds(nbase, HTPS)], degb)

    def disv(v, _):
        d = degb[pl.ds(v * 16, 16)] + 1.0
        yi = jnp.int32(0x5F3759DF) - (lax.bitcast_convert_type(
            d, jnp.int32) >> 1)
        y = lax.bitcast_convert_type(yi, jnp.float32)
        for _it in range(3):
            y = y * (1.5 - 0.5 * d * y * y)
        for l in range(16):
            row = 2 * v + (l // 8)
            col = (l % 8) * 16
            st[row, pl.ds(col, 16)] = jnp.broadcast_to(y[l], (16,))
        return 0

    lax.fori_loop(0, HTPS // 16, disv, 0)
    pltpu.sync_copy(st, dis128.at[pl.ds(nbase // 8, HTPS // 8)])


_deg_kernel = pl.kernel(
    _deg_body,
    out_type=jax.ShapeDtypeStruct((NP8, 128), jnp.float32),
    mesh=_mesh,
    compiler_params=_sc_params,
    scratch_types=(
        [pltpu.VMEM((NR, GWD), jnp.int32),
         pltpu.VMEM((GWD,), jnp.float32),
         pltpu.VMEM((HTPS,), jnp.float32),
         pltpu.VMEM((HTPS // 8, 128), jnp.float32),
         pltpu.VMEM_SHARED((NP,), jnp.float32)]
        + [pltpu.SemaphoreType.DMA] * (2 * NR)
    ),
)


def _gs_body(src3, dst3, xs, zeros2, outa, outb, idxS, idxD,
             r0, r1, r2, r3, acc, *sems):
    rows = (r0, r1, r2, r3)
    isems = sems[:NR]
    dsems = sems[NR:2 * NR]
    gsems = sems[2 * NR:3 * NR]
    ssems = sems[3 * NR:]
    c = lax.axis_index("c")
    s = lax.axis_index("s")
    wid = s * NC + c
    start = wid * GPW

    sl = pl.ds(s * TPS, TPS)
    pltpu.sync_copy(zeros2, acc.at[sl])
    plsc.subcore_barrier()

    def ild(q, g):
        return pltpu.make_async_copy(src3.at[g], idxS.at[q], isems[q])

    def dld(q, g):
        return pltpu.make_async_copy(dst3.at[g], idxD.at[q], dsems[q])

    def gath(q):
        return pltpu.make_async_copy(xs.at[idxS.at[q]], rows[q], gsems[q])

    def scat(q):
        return pltpu.make_async_copy(rows[q], acc.at[idxD.at[q]], ssems[q])

    ild(0, start).start()
    ild(1, start + 1).start()

    def body(i, _):
        for u in range(NR):
            w = (u + 2) % NR
            g = start + i * NR + u

            @pl.when(i >= 1)
            def _():
                scat(u).wait()

            dld(u, g).start()
            ild(u, 0).wait()
            gath(u).start()

            def fire_prev():
                gath(w).wait()
                dld(w, 0).wait()
                pltpu.async_copy(rows[w], acc.at[idxD.at[w]], ssems[w],
                                 add=True)

            if u >= 2:
                fire_prev()

                @pl.when(i <= GPW // NR - 2)
                def _():
                    ild(w, g + 2).start()
            else:
                @pl.when(i >= 1)
                def _():
                    fire_prev()

                ild(w, g + 2).start()
        return 0

    lax.fori_loop(0, GPW // NR, body, 0)
    for u in (2, 3):
        gath(u).wait()
        dld(u, 0).wait()
        pltpu.async_copy(rows[u], acc.at[idxD.at[u]], ssems[u], add=True)
    for u in range(NR):
        scat(u).wait()

    plsc.subcore_barrier()

    @pl.when(c == 0)
    def _w0():
        pltpu.sync_copy(acc.at[sl], outa.at[sl])

    @pl.when(c == 1)
    def _w1():
        pltpu.sync_copy(acc.at[sl], outb.at[sl])


_gs_kernel = pl.kernel(
    _gs_body,
    out_type=(
        jax.ShapeDtypeStruct((NP, D), jnp.float32),
        jax.ShapeDtypeStruct((NP, D), jnp.float32),
    ),
    mesh=_mesh,
    compiler_params=_sc_params,
    scratch_types=(
        [pltpu.VMEM((NR, GW), jnp.int32),
         pltpu.VMEM((NR, GW), jnp.int32)]
        + [pltpu.VMEM((GW, D), jnp.float32)] * NR
        + [pltpu.VMEM_SHARED((NP, D), jnp.float32)]
        + [pltpu.SemaphoreType.DMA] * (4 * NR)
    ),
)


BM8 = 800
GRID = NP8 // BM8

_spec_p = pl.BlockSpec((BM8, 128), lambda i: (i, 0))
_spec_w = pl.BlockSpec((128, 128), lambda i: (0, 0))
_spec_b = pl.BlockSpec((1, 128), lambda i: (0, 0))


def _tc1_body(dis, emb, wbd, xs1):
    h = jnp.dot(emb[...], wbd[...], preferred_element_type=jnp.float32)
    xs1[...] = h * dis[...]


_tc1 = pl.pallas_call(
    _tc1_body,
    grid=(GRID,),
    in_specs=[_spec_p, _spec_p, _spec_w],
    out_specs=_spec_p,
    out_shape=jax.ShapeDtypeStruct((NP8, 128), jnp.float32),
)


def _tc2_body(dis, xs1, acca, accb, wbd, bias, xs2):
    dv = dis[...]
    x2 = jnp.maximum(
        dv * (acca[...] + accb[...] + xs1[...]) + bias[...], 0.0)
    h = jnp.dot(x2, wbd[...], preferred_element_type=jnp.float32)
    xs2[...] = h * dv


_tc2 = pl.pallas_call(
    _tc2_body,
    grid=(GRID,),
    in_specs=[_spec_p, _spec_p, _spec_p, _spec_p, _spec_w, _spec_b],
    out_specs=_spec_p,
    out_shape=jax.ShapeDtypeStruct((NP8, 128), jnp.float32),
)


def _tc3_body(dis, xs2, acca, accb, bias, out):
    out[...] = dis[...] * (acca[...] + accb[...] + xs2[...]) + bias[...]


_tc3 = pl.pallas_call(
    _tc3_body,
    grid=(GRID,),
    in_specs=[_spec_p, _spec_p, _spec_p, _spec_p, _spec_b],
    out_specs=_spec_p,
    out_shape=jax.ShapeDtypeStruct((NP8, 128), jnp.float32),
)


def kernel(edge_index, emb, W1, b1, W2, b2):
    pe = EP - E
    pad_dst = N + (jnp.arange(pe, dtype=jnp.int32) % 2048)
    dstf = jnp.concatenate([edge_index[1], pad_dst])
    zeros1 = jnp.zeros((TPS,), jnp.float32)
    zeros2 = jnp.zeros((TPS, D), jnp.float32)

    dis = _deg_kernel(dstf.reshape(EP // GWD, GWD), zeros1)

    pad_src = (jnp.arange(pe, dtype=jnp.int32) % N)
    src3 = jnp.concatenate([edge_index[0], pad_src]).reshape(GTOT, GW)
    dst3 = dstf.reshape(GTOT, GW)
    eye8 = jnp.eye(8, dtype=jnp.float32)
    wbd1 = jnp.kron(eye8, W1)
    wbd2 = jnp.kron(eye8, W2)
    b1t = jnp.tile(b1, 8).reshape(1, 128)
    b2t = jnp.tile(b2, 8).reshape(1, 128)
    emb8 = jnp.pad(emb.reshape(N // 8, 128), ((0, NP8 - N // 8), (0, 0)))

    xs1 = _tc1(dis, emb8, wbd1)
    a1a, a1b = _gs_kernel(src3, dst3, xs1.reshape(NP, D), zeros2)
    xs2 = _tc2(dis, xs1, a1a.reshape(NP8, 128), a1b.reshape(NP8, 128),
               wbd2, b1t)
    a2a, a2b = _gs_kernel(src3, dst3, xs2.reshape(NP, D), zeros2)
    out = _tc3(dis, xs2, a2a.reshape(NP8, 128), a2b.reshape(NP8, 128), b2t)
    return out[:N // 8].reshape(N, D)


# speedup vs baseline: 160.1931x; 1.1279x over previous
import jax
import jax.numpy as jnp
from jax import lax
from jax.experimental import pallas as pl
from jax.experimental.pallas import tpu as pltpu
from jax.experimental.pallas import tpu_sc as plsc

N = 100000
D = 16
E = 3200000
NP = 102400
TPS = NP // 16
NP8 = NP // 8

NC = 2
NS = 16
NW = NC * NS

GW = 256
GPW = 396
GTOT = NW * GPW
EP = GTOT * GW
NR = 6
NRD = 3
GWD = 2048
GPT = EP // GWD // NS

HTPS = NP // NC // NS

_mesh = plsc.VectorSubcoreMesh(core_axis_name="c", subcore_axis_name="s")
_sc_params = pltpu.CompilerParams(use_tc_tiling_on_sc=False)


def _deg_body(dst3, zeros1, dis128, idxD, ones, degb, st, dacc, *sems):
    dsems = sems[:NRD]
    ssems = sems[NRD:]
    c = lax.axis_index("c")
    s = lax.axis_index("s")

    for k in range(GWD // 16):
        ones[pl.ds(k * 16, 16)] = jnp.ones((16,), jnp.float32)
    sl = pl.ds(s * TPS, TPS)
    pltpu.sync_copy(zeros1, dacc.at[sl])
    plsc.subcore_barrier()

    start = s * GPT

    def dld(q, g):
        return pltpu.make_async_copy(dst3.at[g], idxD.at[q], dsems[q])

    def scat(q):
        return pltpu.make_async_copy(ones, dacc.at[idxD.at[q]], ssems[q])

    def body(i, _):
        for u in range(NRD):
            w = (u + 1) % NRD

            @pl.when(i >= 1)
            def _():
                scat(u).wait()

            dld(u, start + i * NRD + u).start()
            if u == 2:
                dld(w, 0).wait()
                pltpu.async_copy(ones, dacc.at[idxD.at[w]], ssems[w],
                                 add=True)
            else:
                @pl.when(i >= 1)
                def _():
                    dld(w, 0).wait()
                    pltpu.async_copy(ones, dacc.at[idxD.at[w]], ssems[w],
                                     add=True)
        return 0

    lax.fori_loop(0, GPT // NRD, body, 0)
    for u in (1, 2):
        dld(u, 0).wait()
        pltpu.async_copy(ones, dacc.at[idxD.at[u]], ssems[u], add=True)
    for u in range(NRD):
        scat(u).wait()

    plsc.subcore_barrier()

    nbase = (c * NS + s) * HTPS
    pltpu.sync_copy(dacc.at[pl.ds(nbase, HTPS)], degb)

    def disv(v, _):
        d = degb[pl.ds(v * 16, 16)] + 1.0
        yi = jnp.int32(0x5F3759DF) - (lax.bitcast_convert_type(
            d, jnp.int32) >> 1)
        y = lax.bitcast_convert_type(yi, jnp.float32)
        for _it in range(3):
            y = y * (1.5 - 0.5 * d * y * y)
        for l in range(16):
            row = 2 * v + (l // 8)
            col = (l % 8) * 16
            st[row, pl.ds(col, 16)] = jnp.broadcast_to(y[l], (16,))
        return 0

    lax.fori_loop(0, HTPS // 16, disv, 0)
    pltpu.sync_copy(st, dis128.at[pl.ds(nbase // 8, HTPS // 8)])


_deg_kernel = pl.kernel(
    _deg_body,
    out_type=jax.ShapeDtypeStruct((NP8, 128), jnp.float32),
    mesh=_mesh,
    compiler_params=_sc_params,
    scratch_types=(
        [pltpu.VMEM((NRD, GWD), jnp.int32),
         pltpu.VMEM((GWD,), jnp.float32),
         pltpu.VMEM((HTPS,), jnp.float32),
         pltpu.VMEM((HTPS // 8, 128), jnp.float32),
         pltpu.VMEM_SHARED((NP,), jnp.float32)]
        + [pltpu.SemaphoreType.DMA] * (2 * NRD)
    ),
)


def _gs_body(src3, dst3, xs, zeros2, outa, outb, idxS, idxD,
             r0, r1, r2, r3, r4, r5, acc, *sems):
    rows = (r0, r1, r2, r3, r4, r5)
    isems = sems[:NR]
    dsems = sems[NR:2 * NR]
    gsems = sems[2 * NR:3 * NR]
    ssems = sems[3 * NR:]
    c = lax.axis_index("c")
    s = lax.axis_index("s")
    wid = s * NC + c
    start = wid * GPW

    sl = pl.ds(s * TPS, TPS)
    pltpu.sync_copy(zeros2, acc.at[sl])
    plsc.subcore_barrier()

    def ild(q, g):
        return pltpu.make_async_copy(src3.at[g], idxS.at[q], isems[q])

    def dld(q, g):
        return pltpu.make_async_copy(dst3.at[g], idxD.at[q], dsems[q])

    def gath(q):
        return pltpu.make_async_copy(xs.at[idxS.at[q]], rows[q], gsems[q])

    def scat(q):
        return pltpu.make_async_copy(rows[q], acc.at[idxD.at[q]], ssems[q])

    ild(0, start).start()
    ild(1, start + 1).start()
    ild(2, start + 2).start()

    def body(i, _):
        for u in range(NR):
            w = (u + 3) % NR
            g = start + i * NR + u

            @pl.when(i >= 1)
            def _():
                scat(u).wait()

            dld(u, g).start()
            ild(u, 0).wait()
            gath(u).start()

            def fire_prev():
                gath(w).wait()
                dld(w, 0).wait()
                pltpu.async_copy(rows[w], acc.at[idxD.at[w]], ssems[w],
                                 add=True)

            if u >= 3:
                fire_prev()

                @pl.when(i <= GPW // NR - 2)
                def _():
                    ild(w, g + 3).start()
            else:
                @pl.when(i >= 1)
                def _():
                    fire_prev()

                ild(w, g + 3).start()
        return 0

    lax.fori_loop(0, GPW // NR, body, 0)
    for u in (3, 4, 5):
        gath(u).wait()
        dld(u, 0).wait()
        pltpu.async_copy(rows[u], acc.at[idxD.at[u]], ssems[u], add=True)
    for u in range(NR):
        scat(u).wait()

    plsc.subcore_barrier()

    @pl.when(c == 0)
    def _w0():
        pltpu.sync_copy(acc.at[sl], outa.at[sl])

    @pl.when(c == 1)
    def _w1():
        pltpu.sync_copy(acc.at[sl], outb.at[sl])


_gs_kernel = pl.kernel(
    _gs_body,
    out_type=(
        jax.ShapeDtypeStruct((NP, D), jnp.float32),
        jax.ShapeDtypeStruct((NP, D), jnp.float32),
    ),
    mesh=_mesh,
    compiler_params=_sc_params,
    scratch_types=(
        [pltpu.VMEM((NR, GW), jnp.int32),
         pltpu.VMEM((NR, GW), jnp.int32)]
        + [pltpu.VMEM((GW, D), jnp.float32)] * NR
        + [pltpu.VMEM_SHARED((NP, D), jnp.float32)]
        + [pltpu.SemaphoreType.DMA] * (4 * NR)
    ),
)


BM8 = 800
GRID = NP8 // BM8

_spec_p = pl.BlockSpec((BM8, 128), lambda i: (i, 0))
_spec_w = pl.BlockSpec((128, 128), lambda i: (0, 0))
_spec_b = pl.BlockSpec((1, 128), lambda i: (0, 0))


def _tc1_body(dis, emb, wbd, xs1):
    h = jnp.dot(emb[...], wbd[...], preferred_element_type=jnp.float32)
    xs1[...] = h * dis[...]


_tc1 = pl.pallas_call(
    _tc1_body,
    grid=(GRID,),
    in_specs=[_spec_p, _spec_p, _spec_w],
    out_specs=_spec_p,
    out_shape=jax.ShapeDtypeStruct((NP8, 128), jnp.float32),
)


def _tc2_body(dis, xs1, acca, accb, wbd, bias, xs2):
    dv = dis[...]
    x2 = jnp.maximum(
        dv * (acca[...] + accb[...] + xs1[...]) + bias[...], 0.0)
    h = jnp.dot(x2, wbd[...], preferred_element_type=jnp.float32)
    xs2[...] = h * dv


_tc2 = pl.pallas_call(
    _tc2_body,
    grid=(GRID,),
    in_specs=[_spec_p, _spec_p, _spec_p, _spec_p, _spec_w, _spec_b],
    out_specs=_spec_p,
    out_shape=jax.ShapeDtypeStruct((NP8, 128), jnp.float32),
)


def _tc3_body(dis, xs2, acca, accb, bias, out):
    out[...] = dis[...] * (acca[...] + accb[...] + xs2[...]) + bias[...]


_tc3 = pl.pallas_call(
    _tc3_body,
    grid=(GRID,),
    in_specs=[_spec_p, _spec_p, _spec_p, _spec_p, _spec_b],
    out_specs=_spec_p,
    out_shape=jax.ShapeDtypeStruct((NP8, 128), jnp.float32),
)


def kernel(edge_index, emb, W1, b1, W2, b2):
    pe = EP - E
    pad_dst = N + (jnp.arange(pe, dtype=jnp.int32) % 2048)
    dstf = jnp.concatenate([edge_index[1], pad_dst])
    zeros1 = jnp.zeros((TPS,), jnp.float32)
    zeros2 = jnp.zeros((TPS, D), jnp.float32)

    dis = _deg_kernel(dstf.reshape(EP // GWD, GWD), zeros1)

    pad_src = (jnp.arange(pe, dtype=jnp.int32) % N)
    src3 = jnp.concatenate([edge_index[0], pad_src]).reshape(GTOT, GW)
    dst3 = dstf.reshape(GTOT, GW)
    eye8 = jnp.eye(8, dtype=jnp.float32)
    wbd1 = jnp.kron(eye8, W1)
    wbd2 = jnp.kron(eye8, W2)
    b1t = jnp.tile(b1, 8).reshape(1, 128)
    b2t = jnp.tile(b2, 8).reshape(1, 128)
    emb8 = jnp.pad(emb.reshape(N // 8, 128), ((0, NP8 - N // 8), (0, 0)))

    xs1 = _tc1(dis, emb8, wbd1)
    a1a, a1b = _gs_kernel(src3, dst3, xs1.reshape(NP, D), zeros2)
    xs2 = _tc2(dis, xs1, a1a.reshape(NP8, 128), a1b.reshape(NP8, 128),
               wbd2, b1t)
    a2a, a2b = _gs_kernel(src3, dst3, xs2.reshape(NP, D), zeros2)
    out = _tc3(dis, xs2, a2a.reshape(NP8, 128), a2b.reshape(NP8, 128), b2t)
    return out[:N // 8].reshape(N, D)
